# Initial kernel scaffold; baseline (speedup 1.0000x reference)
import jax
import jax.numpy as jnp
from jax.experimental import pallas as pl


def kernel(hidden_states, router_w, w_gate, w_up, w_down):
    raise NotImplementedError("write your pallas kernel here")


# speedup vs baseline: 1.2554x; 1.2554x over previous
import functools

import jax
import jax.numpy as jnp
from jax.experimental import pallas as pl
from jax.experimental.pallas import tpu as pltpu


def _moe_body(x_ref, rw_ref, wg_ref, wu_ref, wd_ref, out_ref, acc_ref):
    e = pl.program_id(0)
    n_e = pl.num_programs(0)
    t = pl.program_id(1)

    xb = x_ref[...]
    tb, d = xb.shape
    num_experts = rw_ref.shape[1]

    logits = jnp.dot(xb, rw_ref[...], preferred_element_type=jnp.float32)
    m = jnp.max(logits, axis=1, keepdims=True)
    p = jnp.exp(logits - m)
    p = p / jnp.sum(p, axis=1, keepdims=True)
    ii = jax.lax.broadcasted_iota(jnp.int32, (tb, num_experts), 1)
    p1 = jnp.max(p, axis=1, keepdims=True)
    i1 = jnp.min(jnp.where(p == p1, ii, num_experts), axis=1, keepdims=True)
    m1 = ii == i1
    pm = jnp.where(m1, -jnp.inf, p)
    p2 = jnp.max(pm, axis=1, keepdims=True)
    i2 = jnp.min(jnp.where(pm == p2, ii, num_experts), axis=1, keepdims=True)
    m2 = ii == i2
    r = jnp.where(m1 | m2, p, 0.0) / (p1 + p2)
    r_e = jnp.sum(jnp.where(ii == e, r, 0.0), axis=1)

    xb16 = xb.astype(jnp.bfloat16)
    wg = wg_ref[0].astype(jnp.bfloat16)
    wu = wu_ref[0].astype(jnp.bfloat16)
    wd = wd_ref[0].astype(jnp.bfloat16)
    g = jnp.dot(xb16, wg, preferred_element_type=jnp.float32)
    u = jnp.dot(xb16, wu, preferred_element_type=jnp.float32)
    h = (g * jax.nn.sigmoid(g)) * u
    y = jnp.dot(h.astype(jnp.bfloat16), wd, preferred_element_type=jnp.float32)

    contrib = r_e[:, None] * y
    tsl = pl.ds(t * tb, tb)

    @pl.when(e == 0)
    def _init():
        acc_ref[tsl, :] = contrib

    @pl.when((e != 0) & (e != n_e - 1))
    def _acc():
        acc_ref[tsl, :] += contrib

    @pl.when(e == n_e - 1)
    def _fin():
        out_ref[...] = acc_ref[tsl, :] + contrib


@functools.partial(jax.jit, static_argnames=("tb",))
def _moe_call(x, rw, wg, wu, wd, tb=256):
    t, d = x.shape
    e_num, _, f = wg.shape
    grid = (e_num, t // tb)
    return pl.pallas_call(
        _moe_body,
        grid=grid,
        in_specs=[
            pl.BlockSpec((tb, d), lambda ei, ti: (ti, 0)),
            pl.BlockSpec((d, e_num), lambda ei, ti: (0, 0)),
            pl.BlockSpec((1, d, f), lambda ei, ti: (ei, 0, 0)),
            pl.BlockSpec((1, d, f), lambda ei, ti: (ei, 0, 0)),
            pl.BlockSpec((1, f, d), lambda ei, ti: (ei, 0, 0)),
        ],
        out_specs=pl.BlockSpec((tb, d), lambda ei, ti: (ti, 0)),
        out_shape=jax.ShapeDtypeStruct((t, d), jnp.float32),
        scratch_shapes=[pltpu.VMEM((t, d), jnp.float32)],
    )(x, rw, wg, wu, wd)


def kernel(hidden_states, router_w, w_gate, w_up, w_down):
    return _moe_call(hidden_states, router_w, w_gate, w_up, w_down)


# speedup vs baseline: 1.4011x; 1.1160x over previous
import functools

import jax
import jax.numpy as jnp
from jax.experimental import pallas as pl
from jax.experimental.pallas import tpu as pltpu


def _moe_body(x_ref, rw_ref, wg_ref, wu_ref, wd_ref, out_ref,
              wg16_ref, wu16_ref, wd16_ref, x16_ref, rt_ref, acc_ref):
    e = pl.program_id(0)
    n_e = pl.num_programs(0)
    t = pl.program_id(1)
    tb = x_ref.shape[0]
    num_experts = rw_ref.shape[1]
    tsl = pl.ds(t * tb, tb)

    @pl.when(t == 0)
    def _cast_w():
        wg16_ref[...] = wg_ref[0].astype(jnp.bfloat16)
        wu16_ref[...] = wu_ref[0].astype(jnp.bfloat16)
        wd16_ref[...] = wd_ref[0].astype(jnp.bfloat16)

    @pl.when(e == 0)
    def _routing():
        xb = x_ref[...]
        x16_ref[tsl, :] = xb.astype(jnp.bfloat16)
        logits = jnp.dot(xb, rw_ref[...], preferred_element_type=jnp.float32)
        m = jnp.max(logits, axis=1, keepdims=True)
        p = jnp.exp(logits - m)
        p = p / jnp.sum(p, axis=1, keepdims=True)
        ii = jax.lax.broadcasted_iota(jnp.int32, (tb, num_experts), 1)
        p1 = jnp.max(p, axis=1, keepdims=True)
        i1 = jnp.min(jnp.where(p == p1, ii, num_experts), axis=1, keepdims=True)
        m1 = ii == i1
        pm = jnp.where(m1, -jnp.inf, p)
        p2 = jnp.max(pm, axis=1, keepdims=True)
        i2 = jnp.min(jnp.where(pm == p2, ii, num_experts), axis=1, keepdims=True)
        m2 = ii == i2
        r = jnp.where(m1 | m2, p, 0.0) / (p1 + p2)
        rt_ref[:, tsl] = r.T

    x16 = x16_ref[tsl, :]
    g = jnp.dot(x16, wg16_ref[...], preferred_element_type=jnp.float32)
    u = jnp.dot(x16, wu16_ref[...], preferred_element_type=jnp.float32)
    h = (g * jax.nn.sigmoid(g)) * u
    y = jnp.dot(h.astype(jnp.bfloat16), wd16_ref[...],
                preferred_element_type=jnp.float32)

    r_e = rt_ref[e, tsl]
    contrib = r_e[:, None] * y

    @pl.when(e == 0)
    def _init():
        acc_ref[tsl, :] = contrib

    @pl.when((e != 0) & (e != n_e - 1))
    def _acc():
        acc_ref[tsl, :] += contrib

    @pl.when(e == n_e - 1)
    def _fin():
        out_ref[...] = acc_ref[tsl, :] + contrib


@functools.partial(jax.jit, static_argnames=("tb",))
def _moe_call(x, rw, wg, wu, wd, tb=256):
    t, d = x.shape
    e_num, _, f = wg.shape
    grid = (e_num, t // tb)
    return pl.pallas_call(
        _moe_body,
        grid=grid,
        in_specs=[
            pl.BlockSpec((tb, d), lambda ei, ti: (ti, 0)),
            pl.BlockSpec((d, e_num), lambda ei, ti: (0, 0)),
            pl.BlockSpec((1, d, f), lambda ei, ti: (ei, 0, 0)),
            pl.BlockSpec((1, d, f), lambda ei, ti: (ei, 0, 0)),
            pl.BlockSpec((1, f, d), lambda ei, ti: (ei, 0, 0)),
        ],
        out_specs=pl.BlockSpec((tb, d), lambda ei, ti: (ti, 0)),
        out_shape=jax.ShapeDtypeStruct((t, d), jnp.float32),
        scratch_shapes=[
            pltpu.VMEM((d, f), jnp.bfloat16),
            pltpu.VMEM((d, f), jnp.bfloat16),
            pltpu.VMEM((f, d), jnp.bfloat16),
            pltpu.VMEM((t, d), jnp.bfloat16),
            pltpu.VMEM((e_num, t), jnp.float32),
            pltpu.VMEM((t, d), jnp.float32),
        ],
    )(x, rw, wg, wu, wd)


def kernel(hidden_states, router_w, w_gate, w_up, w_down):
    return _moe_call(hidden_states, router_w, w_gate, w_up, w_down)
